# baseline (speedup 1.0000x reference)
import functools

import jax
import jax.numpy as jnp
from jax import lax
from jax.experimental import pallas as pl
from jax.experimental.pallas import tpu as pltpu
from jax.experimental.pallas import tpu_sc as plsc

TOKENS = 16384
D_MODEL = 4096
N_EXPERTS = 64
TOP_K = 8
TB = 512

NW = 32
CHUNK_TOKENS = (4096, 4096, 4096, 4096)


def _matmul_body(x_ref, w_ref, b_ref, s_ref):
    z = jnp.dot(x_ref[...], w_ref[...], preferred_element_type=jnp.float32)
    s_ref[...] = jax.nn.sigmoid(z + b_ref[...])


_GDN = lax.GatherDimensionNumbers(
    offset_dims=(), collapsed_slice_dims=(0,), start_index_map=(0,)
)


def _lane_perm(x, perm):
    return lax.gather(
        x, perm[:, None], _GDN, (1,),
        mode=lax.GatherScatterMode.PROMISE_IN_BOUNDS,
    )


def _make_route(ct):
    chunk = ct // NW
    pairs = chunk // 2

    def body(s_hbm, gs_hbm, idx_hbm, s_v, gs_v, idx_v):
        wid = lax.axis_index("s") * 2 + lax.axis_index("c")
        base = wid * chunk
        pltpu.sync_copy(s_hbm.at[pl.ds(base, chunk), :], s_v)

        lane = lax.iota(jnp.int32, 16)
        m8 = lane < 8
        perm = jnp.where(m8, lane, lane - 8)

        def merge(ka, va, kb, vb):
            km = jnp.where(m8, ka, _lane_perm(kb, perm))
            vm = jnp.where(m8, va, _lane_perm(vb, perm))
            return plsc.sort_key_val(km, vm, descending=True)

        def topk_token(t):
            ks, vs = [], []
            for q in range(4):
                k = s_v[t, pl.ds(q * 16, 16)]
                kk, vv = plsc.sort_key_val(k, lane + q * 16, descending=True)
                ks.append(kk)
                vs.append(vv)
            k01, v01 = merge(ks[0], vs[0], ks[1], vs[1])
            k23, v23 = merge(ks[2], vs[2], ks[3], vs[3])
            kf, vf = merge(k01, v01, k23, v23)
            tot = jnp.sum(jnp.where(m8, kf, 0.0))
            return kf / tot, vf

        @plsc.parallel_loop(0, pairs, unroll=4)
        def pair_body(p):
            ga, ia = topk_token(2 * p)
            gb, ib = topk_token(2 * p + 1)
            gs_v[pl.ds(16 * p, 16)] = jnp.where(m8, ga, _lane_perm(gb, perm))
            idx_v[pl.ds(16 * p, 16)] = jnp.where(m8, ia, _lane_perm(ib, perm))

        pltpu.sync_copy(gs_v, gs_hbm.at[pl.ds(base * TOP_K, chunk * TOP_K)])
        pltpu.sync_copy(idx_v, idx_hbm.at[pl.ds(base * TOP_K, chunk * TOP_K)])

    return functools.partial(
        pl.kernel,
        out_type=[
            jax.ShapeDtypeStruct((ct * TOP_K,), jnp.float32),
            jax.ShapeDtypeStruct((ct * TOP_K,), jnp.int32),
        ],
        mesh=plsc.VectorSubcoreMesh(core_axis_name="c", subcore_axis_name="s"),
        scratch_types=[
            pltpu.VMEM((chunk, N_EXPERTS), jnp.float32),
            pltpu.VMEM((chunk * TOP_K,), jnp.float32),
            pltpu.VMEM((chunk * TOP_K,), jnp.int32),
        ],
        compiler_params=pltpu.CompilerParams(needs_layout_passes=False),
    )(body)


_route_calls = {ct: _make_route(ct) for ct in sorted(set(CHUNK_TOKENS))}


def _matmul_chunk(x, W, b2, tok_off, ct):
    return pl.pallas_call(
        _matmul_body,
        grid=(ct // TB,),
        in_specs=[
            pl.BlockSpec(
                (TB, D_MODEL), lambda t, o=tok_off // TB: (t + o, 0)
            ),
            pl.BlockSpec((D_MODEL, N_EXPERTS), lambda t: (0, 0)),
            pl.BlockSpec((1, N_EXPERTS), lambda t: (0, 0)),
        ],
        out_specs=pl.BlockSpec((TB, N_EXPERTS), lambda t: (t, 0)),
        out_shape=jax.ShapeDtypeStruct((ct, N_EXPERTS), jnp.float32),
        compiler_params=pltpu.CompilerParams(
            dimension_semantics=("arbitrary",),
        ),
    )(x, W, b2)


@jax.jit
def kernel(x, W, b):
    b2 = b.reshape(1, N_EXPERTS)
    ss, gss, idxs = [], [], []
    off = 0
    for ct in CHUNK_TOKENS:
        s_c = _matmul_chunk(x, W, b2, off, ct)
        g_c, i_c = _route_calls[ct](s_c)
        ss.append(s_c)
        gss.append(g_c)
        idxs.append(i_c)
        off += ct
    s = jnp.concatenate(ss, axis=0)
    gs = jnp.concatenate(gss, axis=0).reshape(TOKENS, TOP_K)
    idx = jnp.concatenate(idxs, axis=0).reshape(TOKENS, TOP_K)
    return (gs, idx, s)


# speedup vs baseline: 1.0890x; 1.0890x over previous
import functools

import jax
import jax.numpy as jnp
from jax import lax
from jax.experimental import pallas as pl
from jax.experimental.pallas import tpu as pltpu
from jax.experimental.pallas import tpu_sc as plsc

TOKENS = 16384
D_MODEL = 4096
N_EXPERTS = 64
TOP_K = 8
TB = 512

NW = 32
CHUNK_TOKENS = (8192, 8192)


def _matmul_body(x_ref, w_ref, b_ref, s_ref):
    z = jnp.dot(x_ref[...], w_ref[...], preferred_element_type=jnp.float32)
    s_ref[...] = jax.nn.sigmoid(z + b_ref[...])


_GDN = lax.GatherDimensionNumbers(
    offset_dims=(), collapsed_slice_dims=(0,), start_index_map=(0,)
)


def _lane_perm(x, perm):
    return lax.gather(
        x, perm[:, None], _GDN, (1,),
        mode=lax.GatherScatterMode.PROMISE_IN_BOUNDS,
    )


def _make_route(ct):
    chunk = ct // NW
    pairs = chunk // 2

    def body(s_hbm, gs_hbm, idx_hbm, s_v, gs_v, idx_v):
        wid = lax.axis_index("s") * 2 + lax.axis_index("c")
        base = wid * chunk
        pltpu.sync_copy(s_hbm.at[pl.ds(base, chunk), :], s_v)

        lane = lax.iota(jnp.int32, 16)
        m8 = lane < 8
        perm = jnp.where(m8, lane, lane - 8)

        def merge(ka, va, kb, vb):
            km = jnp.where(m8, ka, _lane_perm(kb, perm))
            vm = jnp.where(m8, va, _lane_perm(vb, perm))
            return plsc.sort_key_val(km, vm, descending=True)

        def topk_token(t):
            ks, vs = [], []
            for q in range(4):
                k = s_v[t, pl.ds(q * 16, 16)]
                kk, vv = plsc.sort_key_val(k, lane + q * 16, descending=True)
                ks.append(kk)
                vs.append(vv)
            k01, v01 = merge(ks[0], vs[0], ks[1], vs[1])
            k23, v23 = merge(ks[2], vs[2], ks[3], vs[3])
            kf, vf = merge(k01, v01, k23, v23)
            tot = jnp.sum(jnp.where(m8, kf, 0.0))
            return kf / tot, vf

        @plsc.parallel_loop(0, pairs, unroll=8)
        def pair_body(p):
            ga, ia = topk_token(2 * p)
            gb, ib = topk_token(2 * p + 1)
            gs_v[pl.ds(16 * p, 16)] = jnp.where(m8, ga, _lane_perm(gb, perm))
            idx_v[pl.ds(16 * p, 16)] = jnp.where(m8, ia, _lane_perm(ib, perm))

        pltpu.sync_copy(gs_v, gs_hbm.at[pl.ds(base * TOP_K, chunk * TOP_K)])
        pltpu.sync_copy(idx_v, idx_hbm.at[pl.ds(base * TOP_K, chunk * TOP_K)])

    return functools.partial(
        pl.kernel,
        out_type=[
            jax.ShapeDtypeStruct((ct * TOP_K,), jnp.float32),
            jax.ShapeDtypeStruct((ct * TOP_K,), jnp.int32),
        ],
        mesh=plsc.VectorSubcoreMesh(core_axis_name="c", subcore_axis_name="s"),
        scratch_types=[
            pltpu.VMEM((chunk, N_EXPERTS), jnp.float32),
            pltpu.VMEM((chunk * TOP_K,), jnp.float32),
            pltpu.VMEM((chunk * TOP_K,), jnp.int32),
        ],
        compiler_params=pltpu.CompilerParams(needs_layout_passes=False),
    )(body)


_route_calls = {ct: _make_route(ct) for ct in sorted(set(CHUNK_TOKENS))}


def _matmul_chunk(x, W, b2, tok_off, ct):
    return pl.pallas_call(
        _matmul_body,
        grid=(ct // TB,),
        in_specs=[
            pl.BlockSpec(
                (TB, D_MODEL), lambda t, o=tok_off // TB: (t + o, 0)
            ),
            pl.BlockSpec((D_MODEL, N_EXPERTS), lambda t: (0, 0)),
            pl.BlockSpec((1, N_EXPERTS), lambda t: (0, 0)),
        ],
        out_specs=pl.BlockSpec((TB, N_EXPERTS), lambda t: (t, 0)),
        out_shape=jax.ShapeDtypeStruct((ct, N_EXPERTS), jnp.float32),
        compiler_params=pltpu.CompilerParams(
            dimension_semantics=("arbitrary",),
        ),
    )(x, W, b2)


@jax.jit
def kernel(x, W, b):
    b2 = b.reshape(1, N_EXPERTS)
    ss, gss, idxs = [], [], []
    off = 0
    for ct in CHUNK_TOKENS:
        s_c = _matmul_chunk(x, W, b2, off, ct)
        g_c, i_c = _route_calls[ct](s_c)
        ss.append(s_c)
        gss.append(g_c)
        idxs.append(i_c)
        off += ct
    s = jnp.concatenate(ss, axis=0)
    gs = jnp.concatenate(gss, axis=0).reshape(TOKENS, TOP_K)
    idx = jnp.concatenate(idxs, axis=0).reshape(TOKENS, TOP_K)
    return (gs, idx, s)


# speedup vs baseline: 1.1157x; 1.0246x over previous
import functools

import jax
import jax.numpy as jnp
from jax import lax
from jax.experimental import pallas as pl
from jax.experimental.pallas import tpu as pltpu
from jax.experimental.pallas import tpu_sc as plsc

TOKENS = 16384
D_MODEL = 4096
N_EXPERTS = 64
TOP_K = 8
TB = 512

NW = 32
CHUNK_TOKENS = (16384,)


def _matmul_body(x_ref, w_ref, b_ref, s_ref):
    z = jnp.dot(x_ref[...], w_ref[...], preferred_element_type=jnp.float32)
    s_ref[...] = jax.nn.sigmoid(z + b_ref[...])


_GDN = lax.GatherDimensionNumbers(
    offset_dims=(), collapsed_slice_dims=(0,), start_index_map=(0,)
)


def _lane_perm(x, perm):
    return lax.gather(
        x, perm[:, None], _GDN, (1,),
        mode=lax.GatherScatterMode.PROMISE_IN_BOUNDS,
    )


def _make_route(ct):
    chunk = ct // NW
    pairs = chunk // 2

    def body(s_hbm, gs_hbm, idx_hbm, s_v, gs_v, idx_v):
        wid = lax.axis_index("s") * 2 + lax.axis_index("c")
        base = wid * chunk
        pltpu.sync_copy(s_hbm.at[pl.ds(base, chunk), :], s_v)

        lane = lax.iota(jnp.int32, 16)
        m8 = lane < 8
        perm = jnp.where(m8, lane, lane - 8)

        def merge(ka, va, kb, vb):
            km = jnp.where(m8, ka, _lane_perm(kb, perm))
            vm = jnp.where(m8, va, _lane_perm(vb, perm))
            return plsc.sort_key_val(km, vm, descending=True)

        def topk_token(t):
            ks, vs = [], []
            for q in range(4):
                k = s_v[t, pl.ds(q * 16, 16)]
                kk, vv = plsc.sort_key_val(k, lane + q * 16, descending=True)
                ks.append(kk)
                vs.append(vv)
            k01, v01 = merge(ks[0], vs[0], ks[1], vs[1])
            k23, v23 = merge(ks[2], vs[2], ks[3], vs[3])
            kf, vf = merge(k01, v01, k23, v23)
            tot = jnp.sum(jnp.where(m8, kf, 0.0))
            return kf / tot, vf

        @plsc.parallel_loop(0, pairs, unroll=8)
        def pair_body(p):
            ga, ia = topk_token(2 * p)
            gb, ib = topk_token(2 * p + 1)
            gs_v[pl.ds(16 * p, 16)] = jnp.where(m8, ga, _lane_perm(gb, perm))
            idx_v[pl.ds(16 * p, 16)] = jnp.where(m8, ia, _lane_perm(ib, perm))

        pltpu.sync_copy(gs_v, gs_hbm.at[pl.ds(base * TOP_K, chunk * TOP_K)])
        pltpu.sync_copy(idx_v, idx_hbm.at[pl.ds(base * TOP_K, chunk * TOP_K)])

    return functools.partial(
        pl.kernel,
        out_type=[
            jax.ShapeDtypeStruct((ct * TOP_K,), jnp.float32),
            jax.ShapeDtypeStruct((ct * TOP_K,), jnp.int32),
        ],
        mesh=plsc.VectorSubcoreMesh(core_axis_name="c", subcore_axis_name="s"),
        scratch_types=[
            pltpu.VMEM((chunk, N_EXPERTS), jnp.float32),
            pltpu.VMEM((chunk * TOP_K,), jnp.float32),
            pltpu.VMEM((chunk * TOP_K,), jnp.int32),
        ],
        compiler_params=pltpu.CompilerParams(needs_layout_passes=False),
    )(body)


_route_calls = {ct: _make_route(ct) for ct in sorted(set(CHUNK_TOKENS))}


def _matmul_chunk(x, W, b2, tok_off, ct):
    return pl.pallas_call(
        _matmul_body,
        grid=(ct // TB,),
        in_specs=[
            pl.BlockSpec(
                (TB, D_MODEL), lambda t, o=tok_off // TB: (t + o, 0)
            ),
            pl.BlockSpec((D_MODEL, N_EXPERTS), lambda t: (0, 0)),
            pl.BlockSpec((1, N_EXPERTS), lambda t: (0, 0)),
        ],
        out_specs=pl.BlockSpec((TB, N_EXPERTS), lambda t: (t, 0)),
        out_shape=jax.ShapeDtypeStruct((ct, N_EXPERTS), jnp.float32),
        compiler_params=pltpu.CompilerParams(
            dimension_semantics=("arbitrary",),
        ),
    )(x, W, b2)


@jax.jit
def kernel(x, W, b):
    b2 = b.reshape(1, N_EXPERTS)
    ss, gss, idxs = [], [], []
    off = 0
    for ct in CHUNK_TOKENS:
        s_c = _matmul_chunk(x, W, b2, off, ct)
        g_c, i_c = _route_calls[ct](s_c)
        ss.append(s_c)
        gss.append(g_c)
        idxs.append(i_c)
        off += ct
    s = jnp.concatenate(ss, axis=0)
    gs = jnp.concatenate(gss, axis=0).reshape(TOKENS, TOP_K)
    idx = jnp.concatenate(idxs, axis=0).reshape(TOKENS, TOP_K)
    return (gs, idx, s)
